# baseline (speedup 1.0000x reference)
import functools
import jax
import jax.numpy as jnp
from jax import lax
from jax.experimental import pallas as pl
from jax.experimental.pallas import tpu as pltpu, tpu_sc as plsc

_SIZE = 32000
_SMOOTHING = 0.1
_CONFIDENCE = 1.0 - _SMOOTHING
_EPS = _SMOOTHING / (_SIZE - 2)

_ROWS = 8192
_NC = 2
_NS = 16
_NW = _NC * _NS
_RPW = _ROWS // _NW
_TROWS = 4
_CHUNK = 8
_NDMA = _RPW // _TROWS
_NCHUNK = _NDMA // _CHUNK


def _sc_body(tgt_hbm, out_hbm, tgt_v, row_t, shared_t, idx2, conf2, sem):
    wid = lax.axis_index("s") * _NC + lax.axis_index("c")
    base = wid * _RPW

    pltpu.sync_copy(tgt_hbm.at[pl.ds(base, _RPW)], tgt_v)

    eps_vec = jnp.full((16,), _EPS, jnp.float32)

    def fill_body(j, carry):
        row_t[pl.ds(j * 16, 16)] = eps_vec
        return carry

    lax.fori_loop(0, _SIZE // 16, fill_body, 0)

    @pl.when(lax.axis_index("s") == 0)
    def _publish():
        for k in range(_TROWS):
            pltpu.sync_copy(row_t, shared_t.at[pl.ds(k * _SIZE, _SIZE)])

    plsc.subcore_barrier()

    def round_body(g, carry):
        grp0 = base // _TROWS + g * _CHUNK
        for j in range(_CHUNK):
            off = pl.multiple_of((grp0 + j) * _TROWS * _SIZE, 32000)
            pltpu.make_async_copy(
                shared_t, out_hbm.at[pl.ds(off, _TROWS * _SIZE)], sem
            ).start()
        for j in range(_CHUNK):
            off = pl.multiple_of((grp0 + j) * _TROWS * _SIZE, 32000)
            pltpu.make_async_copy(
                shared_t, out_hbm.at[pl.ds(off, _TROWS * _SIZE)], sem
            ).wait()
        return carry

    lax.fori_loop(0, _NCHUNK, round_body, 0)

    conf_vec = jnp.full((16,), _CONFIDENCE, jnp.float32)
    lane = lax.iota(jnp.int32, 16)
    for j in range(2):
        for t in range(8):
            o = j * 128 + t * 16
            rows16 = base + o + lane
            tgt16 = tgt_v[pl.ds(o, 16)]
            idx2[j, pl.ds(t * 16, 16)] = rows16 * _SIZE + tgt16
            conf2[j, pl.ds(t * 16, 16)] = conf_vec

    for j in range(2):
        pltpu.make_async_copy(
            conf2.at[j], out_hbm.at[idx2.at[j]], sem
        ).start()
    for j in range(2):
        pltpu.make_async_copy(
            conf2.at[j], out_hbm.at[idx2.at[j]], sem
        ).wait()


def kernel(target):
    mesh = plsc.VectorSubcoreMesh(core_axis_name="c", subcore_axis_name="s")
    sc_call = pl.kernel(
        _sc_body,
        out_type=jax.ShapeDtypeStruct((_ROWS * _SIZE,), jnp.float32),
        mesh=mesh,
        scratch_types=[
            pltpu.VMEM((_RPW,), jnp.int32),
            pltpu.VMEM((_SIZE,), jnp.float32),
            pltpu.VMEM_SHARED((_TROWS * _SIZE,), jnp.float32),
            pltpu.VMEM((2, 128), jnp.int32),
            pltpu.VMEM((2, 128), jnp.float32),
            pltpu.SemaphoreType.DMA,
        ],
    )
    out_flat = sc_call(target.astype(jnp.int32))
    return out_flat.reshape(_ROWS, _SIZE)


# speedup vs baseline: 4.3273x; 4.3273x over previous
import jax
import jax.numpy as jnp
from jax.experimental import pallas as pl

_SIZE = 32000
_SMOOTHING = 0.1
_CONFIDENCE = 1.0 - _SMOOTHING
_EPS = _SMOOTHING / (_SIZE - 2)

_ROWS = 8192
_BLOCK_R = 128


def _smooth_kernel(tgt_ref, out_ref):
    tgt = tgt_ref[0, 0, :]
    cols = jax.lax.broadcasted_iota(jnp.int32, (_BLOCK_R, _SIZE), 1)
    out_ref[:, :] = jnp.where(
        cols == tgt[:, None],
        jnp.float32(_CONFIDENCE),
        jnp.float32(_EPS),
    )


def kernel(target):
    nb = _ROWS // _BLOCK_R
    tgt3 = target.astype(jnp.int32).reshape(nb, 1, _BLOCK_R)
    out = pl.pallas_call(
        _smooth_kernel,
        grid=(nb,),
        in_specs=[pl.BlockSpec((1, 1, _BLOCK_R), lambda i: (i, 0, 0))],
        out_specs=pl.BlockSpec((_BLOCK_R, _SIZE), lambda i: (i, 0)),
        out_shape=jax.ShapeDtypeStruct((_ROWS, _SIZE), jnp.float32),
    )(tgt3)
    return out
